# Initial kernel scaffold; baseline (speedup 1.0000x reference)
import functools

import jax
import jax.numpy as jnp
from jax import lax
from jax.experimental import pallas as pl
from jax.experimental.pallas import tpu as pltpu
from jax.experimental.pallas import tpu_sc as plsc

_EMB = 32
_CB = 4
_SP = 80
_SCM = 40
_L = 16


def _sum_rows(ref, slot, base, n, unroll):
    z = jnp.zeros((_L,), jnp.float32)

    def body(jb, accs):
        a0, a1, a2, a3 = accs
        r0 = base + jb * unroll
        for u in range(unroll):
            lo = ref[slot, r0 + u, pl.ds(0, _L)]
            hi = ref[slot, r0 + u, pl.ds(_L, _L)]
            if u % 2 == 0:
                a0 = a0 + lo
                a1 = a1 + hi
            else:
                a2 = a2 + lo
                a3 = a3 + hi
        return (a0, a1, a2, a3)

    a0, a1, a2, a3 = lax.fori_loop(0, n // unroll, body, (z, z, z, z))
    return a0 + a2, a1 + a3


@functools.lru_cache(maxsize=None)
def _build_pool(B, LP, LC):
    info = plsc.get_sparse_core_info()
    nw = info.num_cores * info.num_subcores
    rpw = B // nw
    nch = rpw // _CB
    ps = LP * _CB // _SP
    cs = LC * _CB // _SCM
    assert B % nw == 0 and rpw % _CB == 0 and nch % 2 == 0
    assert LP * _CB % _SP == 0 and LC * _CB % _SCM == 0
    assert LP % 8 == 0 and LC % 10 == 0

    mesh = plsc.VectorSubcoreMesh(core_axis_name="c", subcore_axis_name="s")

    @functools.partial(
        pl.kernel,
        mesh=mesh,
        out_type=jax.ShapeDtypeStruct((B, 2 * _EMB), jnp.float32),
        scratch_types=[
            pltpu.VMEM((2, ps, _SP), jnp.int32),
            pltpu.VMEM((2, cs, _SCM), jnp.int32),
            pltpu.VMEM((2, _CB * LP, _EMB), jnp.float32),
            pltpu.VMEM((2, _CB * LC, _EMB), jnp.float32),
            pltpu.VMEM((rpw, 2 * _EMB), jnp.float32),
            pltpu.SemaphoreType.DMA,
            pltpu.SemaphoreType.DMA,
            pltpu.SemaphoreType.DMA,
            pltpu.SemaphoreType.DMA,
        ],
    )
    def pool(pidx_hbm, cidx_hbm, pt_hbm, ct_hbm, out_hbm,
             pidx_v, cidx_v, prows_v, crows_v, pooled_v, ps0, ps1, cs0, cs1):
        wid = lax.axis_index("s") * info.num_cores + lax.axis_index("c")
        psems = (ps0, ps1)
        csems = (cs0, cs1)

        def fire(slot, c):
            pltpu.sync_copy(pidx_hbm.at[pl.ds(wid * (nch * ps) + c * ps, ps)],
                            pidx_v.at[slot])
            pltpu.sync_copy(cidx_hbm.at[pl.ds(wid * (nch * cs) + c * cs, cs)],
                            cidx_v.at[slot])
            for j in range(ps):
                pltpu.async_copy(pt_hbm.at[pidx_v.at[slot].at[j]],
                                 prows_v.at[slot].at[pl.ds(j * _SP, _SP)],
                                 psems[slot])
            for j in range(cs):
                pltpu.async_copy(ct_hbm.at[cidx_v.at[slot].at[j]],
                                 crows_v.at[slot].at[pl.ds(j * _SCM, _SCM)],
                                 csems[slot])

        def drain(slot):
            pltpu.make_async_copy(pt_hbm.at[pl.ds(0, _CB * LP)],
                                  prows_v.at[slot], psems[slot]).wait()
            pltpu.make_async_copy(ct_hbm.at[pl.ds(0, _CB * LC)],
                                  crows_v.at[slot], csems[slot]).wait()

        def reduce_store(slot, c):
            for i in range(_CB):
                row = c * _CB + i
                plo, phi = _sum_rows(prows_v, slot, i * LP, LP, 8)
                clo, chi = _sum_rows(crows_v, slot, i * LC, LC, 10)
                pooled_v[row, pl.ds(0, _L)] = plo * (1.0 / LP)
                pooled_v[row, pl.ds(_L, _L)] = phi * (1.0 / LP)
                pooled_v[row, pl.ds(2 * _L, _L)] = clo * (1.0 / LC)
                pooled_v[row, pl.ds(3 * _L, _L)] = chi * (1.0 / LC)

        fire(0, 0)

        def body(i, carry):
            c0 = 2 * i
            fire(1, c0 + 1)
            drain(0)
            reduce_store(0, c0)

            @pl.when(i < nch // 2 - 1)
            def _():
                fire(0, c0 + 2)

            drain(1)
            reduce_store(1, c0 + 1)
            return carry

        lax.fori_loop(0, nch // 2, body, jnp.int32(0))
        pltpu.sync_copy(pooled_v, out_hbm.at[pl.ds(wid * rpw, rpw)])

    return pool


def _mlp_body(x_ref, w1t_ref, b1_ref, w2t_ref, b2_ref, o_ref):
    h = jnp.dot(x_ref[...], w1t_ref[...], preferred_element_type=jnp.float32)
    h = jnp.maximum(h + b1_ref[...], 0.0)
    o_ref[...] = jnp.dot(h, w2t_ref[...], preferred_element_type=jnp.float32) + b2_ref[...]


@functools.lru_cache(maxsize=None)
def _build_mlp(B, dj, dh):
    blk = min(B, 2048)
    assert B % blk == 0
    return pl.pallas_call(
        _mlp_body,
        grid=(B // blk,),
        in_specs=[
            pl.BlockSpec((blk, dj), lambda i: (i, 0)),
            pl.BlockSpec((dj, dh), lambda i: (0, 0)),
            pl.BlockSpec((1, dh), lambda i: (0, 0)),
            pl.BlockSpec((dh, 1), lambda i: (0, 0)),
            pl.BlockSpec((1, 1), lambda i: (0, 0)),
        ],
        out_specs=pl.BlockSpec((blk, 1), lambda i: (i, 0)),
        out_shape=jax.ShapeDtypeStruct((B, 1), jnp.float32),
    )


def kernel(protein_input, compound_input, protein_table, compound_table, W1, b1, W2, b2):
    B, LP = protein_input.shape
    LC = compound_input.shape[1]
    pidx = protein_input.astype(jnp.int32).reshape(B * LP // _SP, _SP)
    cidx = compound_input.astype(jnp.int32).reshape(B * LC // _SCM, _SCM)
    pooled = _build_pool(B, LP, LC)(pidx, cidx,
                                    protein_table.astype(jnp.float32),
                                    compound_table.astype(jnp.float32))
    w1t = W1.T.astype(jnp.float32)
    w2t = W2.T.astype(jnp.float32)
    mlp = _build_mlp(B, w1t.shape[0], w1t.shape[1])
    return mlp(pooled, w1t, b1.reshape(1, -1).astype(jnp.float32),
               w2t, b2.reshape(1, 1).astype(jnp.float32))


# speedup vs baseline: 16.6982x; 16.6982x over previous
import functools

import jax
import jax.numpy as jnp
from jax import lax
from jax.experimental import pallas as pl
from jax.experimental.pallas import tpu as pltpu
from jax.experimental.pallas import tpu_sc as plsc

_EMB = 32
_CB = 4
_SP = 80
_SCM = 40
_L = 16


def _sum_rows(ref, slot, base, n, unroll):
    z = jnp.zeros((_L,), jnp.float32)

    def body(jb, accs):
        a0, a1, a2, a3 = accs
        r0 = base + jb * unroll
        for u in range(unroll):
            lo = ref[slot, r0 + u, pl.ds(0, _L)]
            hi = ref[slot, r0 + u, pl.ds(_L, _L)]
            if u % 2 == 0:
                a0 = a0 + lo
                a1 = a1 + hi
            else:
                a2 = a2 + lo
                a3 = a3 + hi
        return (a0, a1, a2, a3)

    a0, a1, a2, a3 = lax.fori_loop(0, n // unroll, body, (z, z, z, z))
    return a0 + a2, a1 + a3


@functools.lru_cache(maxsize=None)
def _build_pool(B, LP, LC):
    info = plsc.get_sparse_core_info()
    nw = info.num_cores * info.num_subcores
    rpw = B // nw
    nch = rpw // _CB
    ps = LP * _CB // _SP
    cs = LC * _CB // _SCM
    assert B % nw == 0 and rpw % _CB == 0 and nch % 2 == 0
    assert LP * _CB % _SP == 0 and LC * _CB % _SCM == 0
    assert LP % 8 == 0 and LC % 10 == 0

    mesh = plsc.VectorSubcoreMesh(core_axis_name="c", subcore_axis_name="s")

    @functools.partial(
        pl.kernel,
        mesh=mesh,
        out_type=jax.ShapeDtypeStruct((B, 2 * _EMB), jnp.float32),
        compiler_params=pltpu.CompilerParams(use_tc_tiling_on_sc=False),
        scratch_types=[
            pltpu.VMEM((2, ps, _SP), jnp.int32),
            pltpu.VMEM((2, cs, _SCM), jnp.int32),
            pltpu.VMEM((2, _CB * LP, _EMB), jnp.float32),
            pltpu.VMEM((2, _CB * LC, _EMB), jnp.float32),
            pltpu.VMEM((rpw, 2 * _EMB), jnp.float32),
            pltpu.SemaphoreType.DMA,
            pltpu.SemaphoreType.DMA,
            pltpu.SemaphoreType.DMA,
            pltpu.SemaphoreType.DMA,
        ],
    )
    def pool(pidx_hbm, cidx_hbm, pt_hbm, ct_hbm, out_hbm,
             pidx_v, cidx_v, prows_v, crows_v, pooled_v, ps0, ps1, cs0, cs1):
        wid = lax.axis_index("s") * info.num_cores + lax.axis_index("c")
        psems = (ps0, ps1)
        csems = (cs0, cs1)

        def fire(slot, c):
            pltpu.sync_copy(pidx_hbm.at[pl.ds(wid * (nch * ps) + c * ps, ps)],
                            pidx_v.at[slot])
            pltpu.sync_copy(cidx_hbm.at[pl.ds(wid * (nch * cs) + c * cs, cs)],
                            cidx_v.at[slot])
            for j in range(ps):
                pltpu.async_copy(pt_hbm.at[pidx_v.at[slot].at[j]],
                                 prows_v.at[slot].at[pl.ds(j * _SP, _SP)],
                                 psems[slot])
            for j in range(cs):
                pltpu.async_copy(ct_hbm.at[cidx_v.at[slot].at[j]],
                                 crows_v.at[slot].at[pl.ds(j * _SCM, _SCM)],
                                 csems[slot])

        def drain(slot):
            pltpu.make_async_copy(pt_hbm.at[pl.ds(0, _CB * LP)],
                                  prows_v.at[slot], psems[slot]).wait()
            pltpu.make_async_copy(ct_hbm.at[pl.ds(0, _CB * LC)],
                                  crows_v.at[slot], csems[slot]).wait()

        def reduce_store(slot, c):
            for i in range(_CB):
                row = c * _CB + i
                plo, phi = _sum_rows(prows_v, slot, i * LP, LP, 8)
                clo, chi = _sum_rows(crows_v, slot, i * LC, LC, 10)
                pooled_v[row, pl.ds(0, _L)] = plo * (1.0 / LP)
                pooled_v[row, pl.ds(_L, _L)] = phi * (1.0 / LP)
                pooled_v[row, pl.ds(2 * _L, _L)] = clo * (1.0 / LC)
                pooled_v[row, pl.ds(3 * _L, _L)] = chi * (1.0 / LC)

        fire(0, 0)

        def body(i, carry):
            c0 = 2 * i
            fire(1, c0 + 1)
            drain(0)
            reduce_store(0, c0)

            @pl.when(i < nch // 2 - 1)
            def _():
                fire(0, c0 + 2)

            drain(1)
            reduce_store(1, c0 + 1)
            return carry

        lax.fori_loop(0, nch // 2, body, jnp.int32(0))
        pltpu.sync_copy(pooled_v, out_hbm.at[pl.ds(wid * rpw, rpw)])

    return pool


def _mlp_body(x_ref, w1t_ref, b1_ref, w2t_ref, b2_ref, o_ref):
    h = jnp.dot(x_ref[...], w1t_ref[...], preferred_element_type=jnp.float32)
    h = jnp.maximum(h + b1_ref[...], 0.0)
    o_ref[...] = jnp.dot(h, w2t_ref[...], preferred_element_type=jnp.float32) + b2_ref[...]


@functools.lru_cache(maxsize=None)
def _build_mlp(B, dj, dh):
    blk = min(B, 2048)
    assert B % blk == 0
    return pl.pallas_call(
        _mlp_body,
        grid=(B // blk,),
        in_specs=[
            pl.BlockSpec((blk, dj), lambda i: (i, 0)),
            pl.BlockSpec((dj, dh), lambda i: (0, 0)),
            pl.BlockSpec((1, dh), lambda i: (0, 0)),
            pl.BlockSpec((dh, 1), lambda i: (0, 0)),
            pl.BlockSpec((1, 1), lambda i: (0, 0)),
        ],
        out_specs=pl.BlockSpec((blk, 1), lambda i: (i, 0)),
        out_shape=jax.ShapeDtypeStruct((B, 1), jnp.float32),
    )


def kernel(protein_input, compound_input, protein_table, compound_table, W1, b1, W2, b2):
    B, LP = protein_input.shape
    LC = compound_input.shape[1]
    pidx = protein_input.astype(jnp.int32).reshape(B * LP // _SP, _SP)
    cidx = compound_input.astype(jnp.int32).reshape(B * LC // _SCM, _SCM)
    pooled = _build_pool(B, LP, LC)(pidx, cidx,
                                    protein_table.astype(jnp.float32),
                                    compound_table.astype(jnp.float32))
    w1t = W1.T.astype(jnp.float32)
    w2t = W2.T.astype(jnp.float32)
    mlp = _build_mlp(B, w1t.shape[0], w1t.shape[1])
    return mlp(pooled, w1t, b1.reshape(1, -1).astype(jnp.float32),
               w2t, b2.reshape(1, 1).astype(jnp.float32))
